# Initial kernel scaffold; baseline (speedup 1.0000x reference)
import jax
import jax.numpy as jnp
from jax.experimental import pallas as pl


def kernel(x, y):
    raise NotImplementedError("write your pallas kernel here")


# speedup vs baseline: 8.2737x; 8.2737x over previous
import jax
import jax.numpy as jnp
from jax.experimental import pallas as pl
from jax.experimental.pallas import tpu as pltpu

_K = 8


def _digamma_pos(c):
    r = jnp.zeros_like(c)
    for i in range(8):
        r = r + 1.0 / (c + jnp.float32(i))
    w = c + jnp.float32(8.0)
    iw = 1.0 / w
    iw2 = iw * iw
    psi_w = jnp.log(w) - 0.5 * iw - iw2 * (
        jnp.float32(1.0 / 12.0)
        - iw2 * (jnp.float32(1.0 / 120.0) - iw2 * jnp.float32(1.0 / 252.0))
    )
    return psi_w - r


def _mi_row_kernel(x_ref, y_ref, out_ref):
    hw = x_ref.shape[-1]
    xr = x_ref[0, 0, :]
    yr = y_ref[0, 0, :]
    zx = jnp.abs(xr.reshape(hw, 1) - xr.reshape(1, hw))
    zy = jnp.abs(yr.reshape(hw, 1) - yr.reshape(1, hw))
    zmax = jnp.maximum(zx, zy)

    inf = jnp.float32(jnp.inf)
    kk = jnp.float32(_K + 1)

    def body(_, carry):
        thresh, cum, eps = carry
        masked = jnp.where(zmax > thresh, zmax, inf)
        m = jnp.min(masked, axis=1, keepdims=True)
        mult = jnp.sum((zmax == m).astype(jnp.float32), axis=1, keepdims=True)
        take = cum < kk
        eps = jnp.where(take, m, eps)
        cum = jnp.where(take, cum + mult, cum)
        return m, cum, eps

    z0 = jnp.zeros((hw, 1), jnp.float32)
    init = (jnp.full((hw, 1), -inf, jnp.float32), z0, z0)
    _, _, eps = jax.lax.fori_loop(0, _K + 1, body, init)

    cx = jnp.sum((zx < eps).astype(jnp.float32), axis=1, keepdims=True)
    cy = jnp.sum((zy < eps).astype(jnp.float32), axis=1, keepdims=True)
    t = _digamma_pos(cx) + _digamma_pos(cy)
    s = jnp.sum(t, axis=0, keepdims=True) * jnp.float32(1.0 / hw)
    out_ref[0] = s


def kernel(x, y):
    B, C, H, W = x.shape
    BC, HW = B * C, H * W
    xv = x.reshape(BC, 1, HW)
    yv = y.reshape(BC, 1, HW)
    sums = pl.pallas_call(
        _mi_row_kernel,
        grid=(BC,),
        in_specs=[
            pl.BlockSpec((1, 1, HW), lambda i: (i, 0, 0)),
            pl.BlockSpec((1, 1, HW), lambda i: (i, 0, 0)),
        ],
        out_specs=pl.BlockSpec((1, 1, 1), lambda i: (i, 0, 0)),
        out_shape=jax.ShapeDtypeStruct((BC, 1, 1), jnp.float32),
        compiler_params=pltpu.CompilerParams(dimension_semantics=("parallel",)),
    )(xv, yv)
    const = _digamma_pos(jnp.float32(_K)) + _digamma_pos(jnp.float32(HW))
    mi = const - sums.reshape(B, C)
    return jnp.maximum(mi, 0.0)


# speedup vs baseline: 10.5991x; 1.2811x over previous
import functools

import jax
import jax.numpy as jnp
from jax import lax
from jax.experimental import pallas as pl
from jax.experimental.pallas import tpu as pltpu
from jax.experimental.pallas import tpu_sc as plsc
from jax.scipy.special import digamma as _jsp_digamma

_K = 8
_L = 16
_SC_ROWS = 128


def _sc_log(w):
    bits = lax.bitcast_convert_type(w, jnp.int32)
    e = lax.shift_right_logical(bits, 23) - 127
    m = lax.bitcast_convert_type(
        jnp.bitwise_or(jnp.bitwise_and(bits, (1 << 23) - 1), 127 << 23),
        jnp.float32,
    )
    big = m > jnp.float32(1.4142135623730951)
    m = jnp.where(big, m * 0.5, m)
    e = (e + jnp.where(big, 1, 0)).astype(jnp.float32)
    s = (m - 1.0) / (m + 1.0)
    s2 = s * s
    series = 2.0 * s * (1.0 + s2 * (
        jnp.float32(1.0 / 3.0) + s2 * (
            jnp.float32(1.0 / 5.0) + s2 * jnp.float32(1.0 / 7.0))))
    return e * jnp.float32(0.6931471805599453) + series


def _sc_digamma(c):
    r = jnp.zeros_like(c)
    for i in range(8):
        r = r + 1.0 / (c + jnp.float32(i))
    w = c + jnp.float32(8.0)
    iw = 1.0 / w
    iw2 = iw * iw
    return _sc_log(w) - 0.5 * iw - iw2 * (
        jnp.float32(1.0 / 12.0)
        - iw2 * (jnp.float32(1.0 / 120.0) - iw2 * jnp.float32(1.0 / 252.0))
    ) - r


def _sc_body(nc, rpw, hw, x_hbm, y_hbm, out_hbm,
             xv, yv, zxv, zyv, outv):
    nv = hw // _L
    wid = lax.axis_index("s") * nc + lax.axis_index("c")
    inf16 = jnp.full((_L,), jnp.inf, jnp.float32)
    z16 = jnp.zeros((_L,), jnp.float32)

    for r in range(rpw):
        row = wid * rpw + r
        pltpu.sync_copy(x_hbm.at[row], xv)
        pltpu.sync_copy(y_hbm.at[row], yv)

        def group_body(g, acc):
            xi = xv[pl.ds(g * _L, _L)]
            yi = yv[pl.ds(g * _L, _L)]

            def dist_body(jg, m):
                xj = xv[pl.ds(jg * _L, _L)]
                yj = yv[pl.ds(jg * _L, _L)]
                for l in range(_L):
                    dx = jnp.abs(xi - jnp.full((_L,), xj[l]))
                    dy = jnp.abs(yi - jnp.full((_L,), yj[l]))
                    zxv[pl.ds((jg * _L + l) * _L, _L)] = dx
                    zyv[pl.ds((jg * _L + l) * _L, _L)] = dy
                    t = jnp.maximum(dx, dy)
                    mm = []
                    for lvl in range(_K + 1):
                        mm.append(jnp.minimum(m[lvl], t))
                        t = jnp.maximum(m[lvl], t)
                    m = tuple(mm)
                return m

            m = lax.fori_loop(0, nv, dist_body, (inf16,) * (_K + 1))
            eps = m[_K]

            def cnt_body(j, carry):
                ax, ay = carry
                dx = zxv[pl.ds(j * _L, _L)]
                dy = zyv[pl.ds(j * _L, _L)]
                ax = ax + jnp.where(dx < eps, 1.0, 0.0)
                ay = ay + jnp.where(dy < eps, 1.0, 0.0)
                return ax, ay

            ax, ay = lax.fori_loop(0, hw, cnt_body, (z16, z16))
            return acc + _sc_digamma(ax) + _sc_digamma(ay)

        acc = lax.fori_loop(0, nv, group_body, z16)
        outv[pl.ds(r * _L, _L)] = acc * jnp.float32(1.0 / hw)

    pltpu.sync_copy(outv, out_hbm.at[wid])


def _sc_sums(xv2, yv2, hw):
    info = plsc.get_sparse_core_info()
    nc, ns = info.num_cores, info.num_subcores
    nw = nc * ns
    rpw = _SC_ROWS // nw
    mesh = plsc.VectorSubcoreMesh(core_axis_name="c", subcore_axis_name="s")
    body = functools.partial(_sc_body, nc, rpw, hw)
    out = pl.kernel(
        body,
        out_type=jax.ShapeDtypeStruct((nw, rpw * _L), jnp.float32),
        mesh=mesh,
        scratch_types=[
            pltpu.VMEM((hw,), jnp.float32),
            pltpu.VMEM((hw,), jnp.float32),
            pltpu.VMEM((hw * _L,), jnp.float32),
            pltpu.VMEM((hw * _L,), jnp.float32),
            pltpu.VMEM((rpw * _L,), jnp.float32),
        ],
    )(xv2, yv2)
    return out.reshape(_SC_ROWS, _L).sum(axis=-1)


def _digamma_pos(c):
    r = jnp.zeros_like(c)
    for i in range(8):
        r = r + 1.0 / (c + jnp.float32(i))
    w = c + jnp.float32(8.0)
    iw = 1.0 / w
    iw2 = iw * iw
    psi_w = jnp.log(w) - 0.5 * iw - iw2 * (
        jnp.float32(1.0 / 12.0)
        - iw2 * (jnp.float32(1.0 / 120.0) - iw2 * jnp.float32(1.0 / 252.0))
    )
    return psi_w - r


def _mi_row_kernel(x_ref, y_ref, out_ref):
    hw = x_ref.shape[-1]
    xr = x_ref[0, 0, :]
    yr = y_ref[0, 0, :]
    zx = jnp.abs(xr.reshape(hw, 1) - xr.reshape(1, hw))
    zy = jnp.abs(yr.reshape(hw, 1) - yr.reshape(1, hw))
    zmax = jnp.maximum(zx, zy)

    inf = jnp.float32(jnp.inf)
    kk = jnp.float32(_K + 1)

    def body(_, carry):
        thresh, cum, eps = carry
        masked = jnp.where(zmax > thresh, zmax, inf)
        m = jnp.min(masked, axis=1, keepdims=True)
        mult = jnp.sum((zmax == m).astype(jnp.float32), axis=1, keepdims=True)
        take = cum < kk
        eps = jnp.where(take, m, eps)
        cum = jnp.where(take, cum + mult, cum)
        return m, cum, eps

    z0 = jnp.zeros((hw, 1), jnp.float32)
    init = (jnp.full((hw, 1), -inf, jnp.float32), z0, z0)
    _, _, eps = jax.lax.fori_loop(0, _K + 1, body, init)

    cx = jnp.sum((zx < eps).astype(jnp.float32), axis=1, keepdims=True)
    cy = jnp.sum((zy < eps).astype(jnp.float32), axis=1, keepdims=True)
    t = _digamma_pos(cx) + _digamma_pos(cy)
    s = jnp.sum(t, axis=0, keepdims=True) * jnp.float32(1.0 / hw)
    out_ref[0] = s


def _tc_sums(xv3, yv3, n_rows, hw, row0):
    return pl.pallas_call(
        _mi_row_kernel,
        grid=(n_rows,),
        in_specs=[
            pl.BlockSpec((1, 1, hw), lambda i: (i + row0, 0, 0)),
            pl.BlockSpec((1, 1, hw), lambda i: (i + row0, 0, 0)),
        ],
        out_specs=pl.BlockSpec((1, 1, 1), lambda i: (i, 0, 0)),
        out_shape=jax.ShapeDtypeStruct((n_rows, 1, 1), jnp.float32),
        compiler_params=pltpu.CompilerParams(dimension_semantics=("parallel",)),
    )(xv3, yv3).reshape(n_rows)


def kernel(x, y):
    B, C, H, W = x.shape
    BC, HW = B * C, H * W
    xv = x.reshape(BC, HW)
    yv = y.reshape(BC, HW)

    parts = []
    if _SC_ROWS:
        parts.append(_sc_sums(xv, yv, HW))
    if _SC_ROWS < BC:
        xv3 = xv.reshape(BC, 1, HW)
        yv3 = yv.reshape(BC, 1, HW)
        parts.append(_tc_sums(xv3, yv3, BC - _SC_ROWS, HW, _SC_ROWS))
    sums = parts[0] if len(parts) == 1 else jnp.concatenate(parts)

    const = _jsp_digamma(jnp.float32(_K)) + _jsp_digamma(jnp.float32(HW))
    mi = const - sums.reshape(B, C)
    return jnp.maximum(mi, 0.0)


# speedup vs baseline: 15.0757x; 1.4224x over previous
import functools

import jax
import jax.numpy as jnp
from jax import lax
from jax.experimental import pallas as pl
from jax.experimental.pallas import tpu as pltpu
from jax.experimental.pallas import tpu_sc as plsc
from jax.scipy.special import digamma as _jsp_digamma

_K = 8
_L = 16
_SC_ROWS = 64


def _sc_log(w):
    bits = lax.bitcast_convert_type(w, jnp.int32)
    e = lax.shift_right_logical(bits, 23) - 127
    m = lax.bitcast_convert_type(
        jnp.bitwise_or(jnp.bitwise_and(bits, (1 << 23) - 1), 127 << 23),
        jnp.float32,
    )
    big = m > jnp.float32(1.4142135623730951)
    m = jnp.where(big, m * 0.5, m)
    e = (e + jnp.where(big, 1, 0)).astype(jnp.float32)
    s = (m - 1.0) / (m + 1.0)
    s2 = s * s
    series = 2.0 * s * (1.0 + s2 * (
        jnp.float32(1.0 / 3.0) + s2 * (
            jnp.float32(1.0 / 5.0) + s2 * jnp.float32(1.0 / 7.0))))
    return e * jnp.float32(0.6931471805599453) + series


def _sc_digamma(c):
    r = jnp.zeros_like(c)
    for i in range(8):
        r = r + 1.0 / (c + jnp.float32(i))
    w = c + jnp.float32(8.0)
    iw = 1.0 / w
    iw2 = iw * iw
    return _sc_log(w) - 0.5 * iw - iw2 * (
        jnp.float32(1.0 / 12.0)
        - iw2 * (jnp.float32(1.0 / 120.0) - iw2 * jnp.float32(1.0 / 252.0))
    ) - r


def _sc_body(nc, rpw, hw, x_hbm, y_hbm, out_hbm,
             xv, yv, zxv, zyv, outv):
    nv = hw // _L
    wid = lax.axis_index("s") * nc + lax.axis_index("c")
    inf16 = jnp.full((_L,), jnp.inf, jnp.float32)
    z16 = jnp.zeros((_L,), jnp.float32)

    for r in range(rpw):
        row = wid * rpw + r
        pltpu.sync_copy(x_hbm.at[row], xv)
        pltpu.sync_copy(y_hbm.at[row], yv)

        def group_body(g, acc):
            xi = xv[pl.ds(g * _L, _L)]
            yi = yv[pl.ds(g * _L, _L)]

            def dist_body(jg, m):
                xj = xv[pl.ds(jg * _L, _L)]
                yj = yv[pl.ds(jg * _L, _L)]
                for l in range(_L):
                    dx = jnp.abs(xi - jnp.full((_L,), xj[l]))
                    dy = jnp.abs(yi - jnp.full((_L,), yj[l]))
                    zxv[pl.ds((jg * _L + l) * _L, _L)] = dx
                    zyv[pl.ds((jg * _L + l) * _L, _L)] = dy
                    t = jnp.maximum(dx, dy)
                    mm = []
                    for lvl in range(_K + 1):
                        mm.append(jnp.minimum(m[lvl], t))
                        t = jnp.maximum(m[lvl], t)
                    m = tuple(mm)
                return m

            m = lax.fori_loop(0, nv, dist_body, (inf16,) * (_K + 1))
            eps = m[_K]

            def cnt_body(j, carry):
                ax, ay = carry
                dx = zxv[pl.ds(j * _L, _L)]
                dy = zyv[pl.ds(j * _L, _L)]
                ax = ax + jnp.where(dx < eps, 1.0, 0.0)
                ay = ay + jnp.where(dy < eps, 1.0, 0.0)
                return ax, ay

            ax, ay = lax.fori_loop(0, hw, cnt_body, (z16, z16))
            return acc + _sc_digamma(ax) + _sc_digamma(ay)

        acc = lax.fori_loop(0, nv, group_body, z16)
        outv[pl.ds(r * _L, _L)] = acc * jnp.float32(1.0 / hw)

    pltpu.sync_copy(outv, out_hbm.at[wid])


def _sc_sums(xv2, yv2, hw):
    info = plsc.get_sparse_core_info()
    nc, ns = info.num_cores, info.num_subcores
    nw = nc * ns
    rpw = _SC_ROWS // nw
    mesh = plsc.VectorSubcoreMesh(core_axis_name="c", subcore_axis_name="s")
    body = functools.partial(_sc_body, nc, rpw, hw)
    out = pl.kernel(
        body,
        out_type=jax.ShapeDtypeStruct((nw, rpw * _L), jnp.float32),
        mesh=mesh,
        scratch_types=[
            pltpu.VMEM((hw,), jnp.float32),
            pltpu.VMEM((hw,), jnp.float32),
            pltpu.VMEM((hw * _L,), jnp.float32),
            pltpu.VMEM((hw * _L,), jnp.float32),
            pltpu.VMEM((rpw * _L,), jnp.float32),
        ],
    )(xv2, yv2)
    return out.reshape(_SC_ROWS, _L).sum(axis=-1)


def _digamma_pos(c):
    r = jnp.zeros_like(c)
    for i in range(8):
        r = r + 1.0 / (c + jnp.float32(i))
    w = c + jnp.float32(8.0)
    iw = 1.0 / w
    iw2 = iw * iw
    psi_w = jnp.log(w) - 0.5 * iw - iw2 * (
        jnp.float32(1.0 / 12.0)
        - iw2 * (jnp.float32(1.0 / 120.0) - iw2 * jnp.float32(1.0 / 252.0))
    )
    return psi_w - r


def _mi_row_kernel(x_ref, y_ref, out_ref):
    hw = x_ref.shape[-1]
    xr = x_ref[0, 0, :]
    yr = y_ref[0, 0, :]
    zx = jnp.abs(xr.reshape(hw, 1) - xr.reshape(1, hw))
    zy = jnp.abs(yr.reshape(hw, 1) - yr.reshape(1, hw))
    zmax = jnp.maximum(zx, zy)

    inf = jnp.float32(jnp.inf)
    kk = jnp.float32(_K + 1)

    def body(_, carry):
        thresh, cum, eps = carry
        masked = jnp.where(zmax > thresh, zmax, inf)
        m = jnp.min(masked, axis=1, keepdims=True)
        mult = jnp.sum((zmax == m).astype(jnp.float32), axis=1, keepdims=True)
        take = cum < kk
        eps = jnp.where(take, m, eps)
        cum = jnp.where(take, cum + mult, cum)
        return m, cum, eps

    z0 = jnp.zeros((hw, 1), jnp.float32)
    init = (jnp.full((hw, 1), -inf, jnp.float32), z0, z0)
    _, _, eps = jax.lax.fori_loop(0, _K + 1, body, init)

    cx = jnp.sum((zx < eps).astype(jnp.float32), axis=1, keepdims=True)
    cy = jnp.sum((zy < eps).astype(jnp.float32), axis=1, keepdims=True)
    t = _digamma_pos(cx) + _digamma_pos(cy)
    s = jnp.sum(t, axis=0, keepdims=True) * jnp.float32(1.0 / hw)
    out_ref[0] = s


def _tc_sums(xv3, yv3, n_rows, hw, row0):
    return pl.pallas_call(
        _mi_row_kernel,
        grid=(n_rows,),
        in_specs=[
            pl.BlockSpec((1, 1, hw), lambda i: (i + row0, 0, 0)),
            pl.BlockSpec((1, 1, hw), lambda i: (i + row0, 0, 0)),
        ],
        out_specs=pl.BlockSpec((1, 1, 1), lambda i: (i, 0, 0)),
        out_shape=jax.ShapeDtypeStruct((n_rows, 1, 1), jnp.float32),
        compiler_params=pltpu.CompilerParams(dimension_semantics=("parallel",)),
    )(xv3, yv3).reshape(n_rows)


def kernel(x, y):
    B, C, H, W = x.shape
    BC, HW = B * C, H * W
    xv = x.reshape(BC, HW)
    yv = y.reshape(BC, HW)

    parts = []
    if _SC_ROWS:
        parts.append(_sc_sums(xv, yv, HW))
    if _SC_ROWS < BC:
        xv3 = xv.reshape(BC, 1, HW)
        yv3 = yv.reshape(BC, 1, HW)
        parts.append(_tc_sums(xv3, yv3, BC - _SC_ROWS, HW, _SC_ROWS))
    sums = parts[0] if len(parts) == 1 else jnp.concatenate(parts)

    const = _jsp_digamma(jnp.float32(_K)) + _jsp_digamma(jnp.float32(HW))
    mi = const - sums.reshape(B, C)
    return jnp.maximum(mi, 0.0)


# speedup vs baseline: 15.8592x; 1.0520x over previous
import functools

import jax
import jax.numpy as jnp
from jax import lax
from jax.experimental import pallas as pl
from jax.experimental.pallas import tpu as pltpu
from jax.experimental.pallas import tpu_sc as plsc
from jax.scipy.special import digamma as _jsp_digamma

_K = 8
_L = 16
_SC_ROWS = 64


def _sc_log(w):
    bits = lax.bitcast_convert_type(w, jnp.int32)
    e = lax.shift_right_logical(bits, 23) - 127
    m = lax.bitcast_convert_type(
        jnp.bitwise_or(jnp.bitwise_and(bits, (1 << 23) - 1), 127 << 23),
        jnp.float32,
    )
    big = m > jnp.float32(1.4142135623730951)
    m = jnp.where(big, m * 0.5, m)
    e = (e + jnp.where(big, 1, 0)).astype(jnp.float32)
    s = (m - 1.0) / (m + 1.0)
    s2 = s * s
    series = 2.0 * s * (1.0 + s2 * (
        jnp.float32(1.0 / 3.0) + s2 * (
            jnp.float32(1.0 / 5.0) + s2 * jnp.float32(1.0 / 7.0))))
    return e * jnp.float32(0.6931471805599453) + series


def _sc_digamma(c):
    r = jnp.zeros_like(c)
    for i in range(8):
        r = r + 1.0 / (c + jnp.float32(i))
    w = c + jnp.float32(8.0)
    iw = 1.0 / w
    iw2 = iw * iw
    return _sc_log(w) - 0.5 * iw - iw2 * (
        jnp.float32(1.0 / 12.0)
        - iw2 * (jnp.float32(1.0 / 120.0) - iw2 * jnp.float32(1.0 / 252.0))
    ) - r


def _sc_body(nc, rpw, hw, x_hbm, y_hbm, out_hbm,
             xv, yv, zxv, zyv, outv):
    nv = hw // _L
    wid = lax.axis_index("s") * nc + lax.axis_index("c")
    inf16 = jnp.full((_L,), jnp.inf, jnp.float32)
    z16 = jnp.zeros((_L,), jnp.float32)

    for r in range(rpw):
        row = wid * rpw + r
        pltpu.sync_copy(x_hbm.at[row], xv)
        pltpu.sync_copy(y_hbm.at[row], yv)

        def group_body(g, acc):
            xi = xv[pl.ds(g * _L, _L)]
            yi = yv[pl.ds(g * _L, _L)]

            def dist_body(jg, m):
                xj = xv[pl.ds(jg * _L, _L)]
                yj = yv[pl.ds(jg * _L, _L)]
                for l in range(_L):
                    dx = jnp.abs(xi - jnp.full((_L,), xj[l]))
                    dy = jnp.abs(yi - jnp.full((_L,), yj[l]))
                    zxv[pl.ds((jg * _L + l) * _L, _L)] = dx
                    zyv[pl.ds((jg * _L + l) * _L, _L)] = dy
                    t = jnp.maximum(dx, dy)
                    mm = []
                    for lvl in range(_K + 1):
                        mm.append(jnp.minimum(m[lvl], t))
                        t = jnp.maximum(m[lvl], t)
                    m = tuple(mm)
                return m

            m = lax.fori_loop(0, nv, dist_body, (inf16,) * (_K + 1))
            eps = m[_K]

            def cnt_body(j, carry):
                ax, ay = carry
                dx = zxv[pl.ds(j * _L, _L)]
                dy = zyv[pl.ds(j * _L, _L)]
                ax = ax + jnp.where(dx < eps, 1.0, 0.0)
                ay = ay + jnp.where(dy < eps, 1.0, 0.0)
                return ax, ay

            ax, ay = lax.fori_loop(0, hw, cnt_body, (z16, z16))
            return acc + _sc_digamma(ax) + _sc_digamma(ay)

        acc = lax.fori_loop(0, nv, group_body, z16)
        outv[pl.ds(r * _L, _L)] = acc * jnp.float32(1.0 / hw)

    pltpu.sync_copy(outv, out_hbm.at[wid])


def _sc_sums(xv2, yv2, hw):
    info = plsc.get_sparse_core_info()
    nc, ns = info.num_cores, info.num_subcores
    nw = nc * ns
    rpw = _SC_ROWS // nw
    mesh = plsc.VectorSubcoreMesh(core_axis_name="c", subcore_axis_name="s")
    body = functools.partial(_sc_body, nc, rpw, hw)
    out = pl.kernel(
        body,
        out_type=jax.ShapeDtypeStruct((nw, rpw * _L), jnp.float32),
        mesh=mesh,
        scratch_types=[
            pltpu.VMEM((hw,), jnp.float32),
            pltpu.VMEM((hw,), jnp.float32),
            pltpu.VMEM((hw * _L,), jnp.float32),
            pltpu.VMEM((hw * _L,), jnp.float32),
            pltpu.VMEM((rpw * _L,), jnp.float32),
        ],
    )(xv2, yv2)
    return out.reshape(_SC_ROWS, _L).sum(axis=-1)


def _digamma_pos(c):
    r = jnp.zeros_like(c)
    for i in range(8):
        r = r + 1.0 / (c + jnp.float32(i))
    w = c + jnp.float32(8.0)
    iw = 1.0 / w
    iw2 = iw * iw
    psi_w = jnp.log(w) - 0.5 * iw - iw2 * (
        jnp.float32(1.0 / 12.0)
        - iw2 * (jnp.float32(1.0 / 120.0) - iw2 * jnp.float32(1.0 / 252.0))
    )
    return psi_w - r


def _mi_row_kernel(x_ref, y_ref, out_ref):
    hw = x_ref.shape[-1]
    xr = x_ref[0, 0, :]
    yr = y_ref[0, 0, :]
    zx = jnp.abs(xr.reshape(hw, 1) - xr.reshape(1, hw))
    zy = jnp.abs(yr.reshape(hw, 1) - yr.reshape(1, hw))
    zmax = jnp.maximum(zx, zy)

    inf = jnp.float32(jnp.inf)
    kk = jnp.float32(_K + 1)

    def body(_, carry):
        thresh, cum, eps = carry
        masked = jnp.where(zmax > thresh, zmax, inf)
        m = jnp.min(masked, axis=1, keepdims=True)
        mult = jnp.sum((zmax == m).astype(jnp.float32), axis=1, keepdims=True)
        take = cum < kk
        eps = jnp.where(take, m, eps)
        cum = jnp.where(take, cum + mult, cum)
        return m, cum, eps

    z0 = jnp.zeros((hw, 1), jnp.float32)
    cum0 = jnp.sum((zmax == 0.0).astype(jnp.float32), axis=1, keepdims=True)
    _, _, eps = jax.lax.fori_loop(0, _K, body, (z0, cum0, z0))

    cx = jnp.sum((zx < eps).astype(jnp.float32), axis=1, keepdims=True)
    cy = jnp.sum((zy < eps).astype(jnp.float32), axis=1, keepdims=True)
    t = _digamma_pos(cx) + _digamma_pos(cy)
    s = jnp.sum(t, axis=0, keepdims=True) * jnp.float32(1.0 / hw)
    out_ref[0] = s


def _tc_sums(xv3, yv3, n_rows, hw, row0):
    return pl.pallas_call(
        _mi_row_kernel,
        grid=(n_rows,),
        in_specs=[
            pl.BlockSpec((1, 1, hw), lambda i: (i + row0, 0, 0)),
            pl.BlockSpec((1, 1, hw), lambda i: (i + row0, 0, 0)),
        ],
        out_specs=pl.BlockSpec((1, 1, 1), lambda i: (i, 0, 0)),
        out_shape=jax.ShapeDtypeStruct((n_rows, 1, 1), jnp.float32),
        compiler_params=pltpu.CompilerParams(dimension_semantics=("parallel",)),
    )(xv3, yv3).reshape(n_rows)


def kernel(x, y):
    B, C, H, W = x.shape
    BC, HW = B * C, H * W
    xv = x.reshape(BC, HW)
    yv = y.reshape(BC, HW)

    parts = []
    if _SC_ROWS:
        parts.append(_sc_sums(xv, yv, HW))
    if _SC_ROWS < BC:
        xv3 = xv.reshape(BC, 1, HW)
        yv3 = yv.reshape(BC, 1, HW)
        parts.append(_tc_sums(xv3, yv3, BC - _SC_ROWS, HW, _SC_ROWS))
    sums = parts[0] if len(parts) == 1 else jnp.concatenate(parts)

    const = _jsp_digamma(jnp.float32(_K)) + _jsp_digamma(jnp.float32(HW))
    mi = const - sums.reshape(B, C)
    return jnp.maximum(mi, 0.0)


# speedup vs baseline: 20.0103x; 1.2617x over previous
import functools

import jax
import jax.numpy as jnp
from jax import lax
from jax.experimental import pallas as pl
from jax.experimental.pallas import tpu as pltpu
from jax.experimental.pallas import tpu_sc as plsc
from jax.scipy.special import digamma as _jsp_digamma

_K = 8
_L = 16
_SC_ROWS = 64


def _sc_log(w):
    bits = lax.bitcast_convert_type(w, jnp.int32)
    e = lax.shift_right_logical(bits, 23) - 127
    m = lax.bitcast_convert_type(
        jnp.bitwise_or(jnp.bitwise_and(bits, (1 << 23) - 1), 127 << 23),
        jnp.float32,
    )
    big = m > jnp.float32(1.4142135623730951)
    m = jnp.where(big, m * 0.5, m)
    e = (e + jnp.where(big, 1, 0)).astype(jnp.float32)
    s = (m - 1.0) / (m + 1.0)
    s2 = s * s
    series = 2.0 * s * (1.0 + s2 * (
        jnp.float32(1.0 / 3.0) + s2 * (
            jnp.float32(1.0 / 5.0) + s2 * jnp.float32(1.0 / 7.0))))
    return e * jnp.float32(0.6931471805599453) + series


def _sc_digamma(c):
    r = jnp.zeros_like(c)
    for i in range(8):
        r = r + 1.0 / (c + jnp.float32(i))
    w = c + jnp.float32(8.0)
    iw = 1.0 / w
    iw2 = iw * iw
    return _sc_log(w) - 0.5 * iw - iw2 * (
        jnp.float32(1.0 / 12.0)
        - iw2 * (jnp.float32(1.0 / 120.0) - iw2 * jnp.float32(1.0 / 252.0))
    ) - r


def _sc_body(nc, rpw, hw, x_hbm, y_hbm, out_hbm,
             xv, yv, zxv, zyv, outv):
    nv = hw // _L
    wid = lax.axis_index("s") * nc + lax.axis_index("c")
    inf16 = jnp.full((_L,), jnp.inf, jnp.float32)
    z16 = jnp.zeros((_L,), jnp.float32)

    for r in range(rpw):
        row = wid * rpw + r
        pltpu.sync_copy(x_hbm.at[row], xv)
        pltpu.sync_copy(y_hbm.at[row], yv)

        def group_body(g, acc):
            xi = xv[pl.ds(g * _L, _L)]
            yi = yv[pl.ds(g * _L, _L)]

            def dist_body(jg, m):
                xj = xv[pl.ds(jg * _L, _L)]
                yj = yv[pl.ds(jg * _L, _L)]
                for l in range(_L):
                    dx = jnp.abs(xi - jnp.full((_L,), xj[l]))
                    dy = jnp.abs(yi - jnp.full((_L,), yj[l]))
                    zxv[pl.ds((jg * _L + l) * _L, _L)] = dx
                    zyv[pl.ds((jg * _L + l) * _L, _L)] = dy
                    t = jnp.maximum(dx, dy)
                    mm = []
                    for lvl in range(_K + 1):
                        mm.append(jnp.minimum(m[lvl], t))
                        t = jnp.maximum(m[lvl], t)
                    m = tuple(mm)
                return m

            m = lax.fori_loop(0, nv, dist_body, (inf16,) * (_K + 1))
            eps = m[_K]

            def cnt_body(j, carry):
                ax, ay = carry
                dx = zxv[pl.ds(j * _L, _L)]
                dy = zyv[pl.ds(j * _L, _L)]
                ax = ax + jnp.where(dx < eps, 1.0, 0.0)
                ay = ay + jnp.where(dy < eps, 1.0, 0.0)
                return ax, ay

            ax, ay = lax.fori_loop(0, hw, cnt_body, (z16, z16))
            return acc + _sc_digamma(ax) + _sc_digamma(ay)

        acc = lax.fori_loop(0, nv, group_body, z16)
        outv[pl.ds(r * _L, _L)] = acc * jnp.float32(1.0 / hw)

    pltpu.sync_copy(outv, out_hbm.at[wid])


def _sc_sums(xv2, yv2, hw):
    info = plsc.get_sparse_core_info()
    nc, ns = info.num_cores, info.num_subcores
    nw = nc * ns
    rpw = _SC_ROWS // nw
    mesh = plsc.VectorSubcoreMesh(core_axis_name="c", subcore_axis_name="s")
    body = functools.partial(_sc_body, nc, rpw, hw)
    out = pl.kernel(
        body,
        out_type=jax.ShapeDtypeStruct((nw, rpw * _L), jnp.float32),
        mesh=mesh,
        scratch_types=[
            pltpu.VMEM((hw,), jnp.float32),
            pltpu.VMEM((hw,), jnp.float32),
            pltpu.VMEM((hw * _L,), jnp.float32),
            pltpu.VMEM((hw * _L,), jnp.float32),
            pltpu.VMEM((rpw * _L,), jnp.float32),
        ],
    )(xv2, yv2)
    return out.reshape(_SC_ROWS, _L).sum(axis=-1)


def _digamma_pos(c):
    r = jnp.zeros_like(c)
    for i in range(8):
        r = r + 1.0 / (c + jnp.float32(i))
    w = c + jnp.float32(8.0)
    iw = 1.0 / w
    iw2 = iw * iw
    psi_w = jnp.log(w) - 0.5 * iw - iw2 * (
        jnp.float32(1.0 / 12.0)
        - iw2 * (jnp.float32(1.0 / 120.0) - iw2 * jnp.float32(1.0 / 252.0))
    )
    return psi_w - r


def _mi_row_kernel(xc_ref, yc_ref, xr_ref, yr_ref, out_ref):
    hw = xr_ref.shape[-1]
    xr = xr_ref[0]
    yr = yr_ref[0]
    xc = xc_ref[0]
    yc = yc_ref[0]
    zx = jnp.abs(xc - xr)
    zy = jnp.abs(yc - yr)
    zmax = jnp.maximum(zx, zy)

    inf = jnp.float32(jnp.inf)
    kk = jnp.float32(_K + 1)

    def body(_, carry):
        thresh, cum, eps = carry
        masked = jnp.where(zmax > thresh, zmax, inf)
        m = jnp.min(masked, axis=0, keepdims=True)
        mult = jnp.sum((zmax == m).astype(jnp.float32), axis=0, keepdims=True)
        take = cum < kk
        eps = jnp.where(take, m, eps)
        cum = jnp.where(take, cum + mult, cum)
        return m, cum, eps

    z0 = jnp.zeros((1, hw), jnp.float32)
    cum0 = jnp.sum((zmax == 0.0).astype(jnp.float32), axis=0, keepdims=True)
    thresh, cum, eps = jax.lax.fori_loop(0, _K - 1, body, (z0, cum0, z0))
    masked = jnp.where(zmax > thresh, zmax, inf)
    m = jnp.min(masked, axis=0, keepdims=True)
    eps = jnp.where(cum < kk, m, eps)

    cx = jnp.sum((zx < eps).astype(jnp.float32), axis=0, keepdims=True)
    cy = jnp.sum((zy < eps).astype(jnp.float32), axis=0, keepdims=True)
    t = _digamma_pos(cx) + _digamma_pos(cy)
    s = jnp.sum(t, axis=1, keepdims=True) * jnp.float32(1.0 / hw)
    out_ref[0] = s


def _tc_sums(xcol, ycol, xrow, yrow, n_rows, hw, row0):
    return pl.pallas_call(
        _mi_row_kernel,
        grid=(n_rows,),
        in_specs=[
            pl.BlockSpec((1, hw, 1), lambda i: (i + row0, 0, 0)),
            pl.BlockSpec((1, hw, 1), lambda i: (i + row0, 0, 0)),
            pl.BlockSpec((1, 1, hw), lambda i: (i + row0, 0, 0)),
            pl.BlockSpec((1, 1, hw), lambda i: (i + row0, 0, 0)),
        ],
        out_specs=pl.BlockSpec((1, 1, 1), lambda i: (i, 0, 0)),
        out_shape=jax.ShapeDtypeStruct((n_rows, 1, 1), jnp.float32),
        compiler_params=pltpu.CompilerParams(dimension_semantics=("parallel",)),
    )(xcol, ycol, xrow, yrow).reshape(n_rows)


def kernel(x, y):
    B, C, H, W = x.shape
    BC, HW = B * C, H * W
    xv = x.reshape(BC, HW)
    yv = y.reshape(BC, HW)

    parts = []
    if _SC_ROWS:
        parts.append(_sc_sums(xv, yv, HW))
    if _SC_ROWS < BC:
        xcol = xv.reshape(BC, HW, 1)
        ycol = yv.reshape(BC, HW, 1)
        xrow = xv.reshape(BC, 1, HW)
        yrow = yv.reshape(BC, 1, HW)
        parts.append(
            _tc_sums(xcol, ycol, xrow, yrow, BC - _SC_ROWS, HW, _SC_ROWS))
    sums = parts[0] if len(parts) == 1 else jnp.concatenate(parts)

    const = _jsp_digamma(jnp.float32(_K)) + _jsp_digamma(jnp.float32(HW))
    mi = const - sums.reshape(B, C)
    return jnp.maximum(mi, 0.0)


# speedup vs baseline: 22.3474x; 1.1168x over previous
import functools

import jax
import jax.numpy as jnp
from jax import lax
from jax.experimental import pallas as pl
from jax.experimental.pallas import tpu as pltpu
from jax.experimental.pallas import tpu_sc as plsc
from jax.scipy.special import digamma as _jsp_digamma

_K = 8
_L = 16
_SC_ROWS = 64


def _sc_log(w):
    bits = lax.bitcast_convert_type(w, jnp.int32)
    e = lax.shift_right_logical(bits, 23) - 127
    m = lax.bitcast_convert_type(
        jnp.bitwise_or(jnp.bitwise_and(bits, (1 << 23) - 1), 127 << 23),
        jnp.float32,
    )
    big = m > jnp.float32(1.4142135623730951)
    m = jnp.where(big, m * 0.5, m)
    e = (e + jnp.where(big, 1, 0)).astype(jnp.float32)
    s = (m - 1.0) / (m + 1.0)
    s2 = s * s
    series = 2.0 * s * (1.0 + s2 * (
        jnp.float32(1.0 / 3.0) + s2 * (
            jnp.float32(1.0 / 5.0) + s2 * jnp.float32(1.0 / 7.0))))
    return e * jnp.float32(0.6931471805599453) + series


def _sc_digamma(c):
    r = jnp.zeros_like(c)
    for i in range(8):
        r = r + 1.0 / (c + jnp.float32(i))
    w = c + jnp.float32(8.0)
    iw = 1.0 / w
    iw2 = iw * iw
    return _sc_log(w) - 0.5 * iw - iw2 * (
        jnp.float32(1.0 / 12.0)
        - iw2 * (jnp.float32(1.0 / 120.0) - iw2 * jnp.float32(1.0 / 252.0))
    ) - r


_PTS = 4


def _sc_body(nc, rpw, hw, x_hbm, y_hbm, out_hbm, xv, yv, outv):
    nv = hw // _L
    wid = lax.axis_index("s") * nc + lax.axis_index("c")
    inf16 = jnp.full((_L,), jnp.inf, jnp.float32)
    z16 = jnp.zeros((_L,), jnp.float32)
    nsub = _L // _PTS

    for r in range(rpw):
        row = wid * rpw + r
        pltpu.sync_copy(x_hbm.at[row], xv)
        pltpu.sync_copy(y_hbm.at[row], yv)

        def super_body(sb, acc):
            cxa = z16
            cya = z16
            xsb = xv[pl.ds(sb * _L, _L)]
            ysb = yv[pl.ds(sb * _L, _L)]
            for s in range(nsub):
                xs = [xsb[s * _PTS + i] for i in range(_PTS)]
                ys = [ysb[s * _PTS + i] for i in range(_PTS)]

                def dist_body(jg, tops):
                    xj = xv[pl.ds(jg * _L, _L)]
                    yj = yv[pl.ds(jg * _L, _L)]
                    new = []
                    for i in range(_PTS):
                        t = jnp.maximum(jnp.abs(xj - xs[i]),
                                        jnp.abs(yj - ys[i]))
                        lo = jnp.minimum(plsc.sort_key_val(t, t)[0], tops[i])
                        new.append(plsc.sort_key_val(lo, lo,
                                                     descending=True)[0])
                    return tuple(new)

                tops = lax.fori_loop(0, nv, dist_body, (inf16,) * _PTS)
                eps = [tops[i][_L - 1 - _K] for i in range(_PTS)]

                def cnt_body(jg, carry):
                    xj = xv[pl.ds(jg * _L, _L)]
                    yj = yv[pl.ds(jg * _L, _L)]
                    out = []
                    for i in range(_PTS):
                        ax, ay = carry[2 * i], carry[2 * i + 1]
                        out.append(ax + jnp.where(
                            jnp.abs(xj - xs[i]) < eps[i], 1.0, 0.0))
                        out.append(ay + jnp.where(
                            jnp.abs(yj - ys[i]) < eps[i], 1.0, 0.0))
                    return tuple(out)

                cnts = lax.fori_loop(0, nv, cnt_body, (z16,) * (2 * _PTS))
                for i in range(_PTS):
                    onehot = (lax.iota(jnp.int32, _L) == s * _PTS + i)
                    cxa = cxa + jnp.where(onehot, jnp.sum(cnts[2 * i]), 0.0)
                    cya = cya + jnp.where(onehot, jnp.sum(cnts[2 * i + 1]),
                                          0.0)
            return acc + _sc_digamma(cxa) + _sc_digamma(cya)

        acc = lax.fori_loop(0, nv, super_body, z16)
        outv[pl.ds(r * _L, _L)] = acc * jnp.float32(1.0 / hw)

    pltpu.sync_copy(outv, out_hbm.at[wid])


def _sc_sums(xv2, yv2, hw):
    info = plsc.get_sparse_core_info()
    nc, ns = info.num_cores, info.num_subcores
    nw = nc * ns
    rpw = _SC_ROWS // nw
    mesh = plsc.VectorSubcoreMesh(core_axis_name="c", subcore_axis_name="s")
    body = functools.partial(_sc_body, nc, rpw, hw)
    out = pl.kernel(
        body,
        out_type=jax.ShapeDtypeStruct((nw, rpw * _L), jnp.float32),
        mesh=mesh,
        scratch_types=[
            pltpu.VMEM((hw,), jnp.float32),
            pltpu.VMEM((hw,), jnp.float32),
            pltpu.VMEM((rpw * _L,), jnp.float32),
        ],
        compiler_params=pltpu.CompilerParams(needs_layout_passes=False),
    )(xv2, yv2)
    return out.reshape(_SC_ROWS, _L).sum(axis=-1)


def _digamma_pos(c):
    r = jnp.zeros_like(c)
    for i in range(8):
        r = r + 1.0 / (c + jnp.float32(i))
    w = c + jnp.float32(8.0)
    iw = 1.0 / w
    iw2 = iw * iw
    psi_w = jnp.log(w) - 0.5 * iw - iw2 * (
        jnp.float32(1.0 / 12.0)
        - iw2 * (jnp.float32(1.0 / 120.0) - iw2 * jnp.float32(1.0 / 252.0))
    )
    return psi_w - r


def _mi_row_kernel(xc_ref, yc_ref, xr_ref, yr_ref, out_ref):
    hw = xr_ref.shape[-1]
    xr = xr_ref[0]
    yr = yr_ref[0]
    xc = xc_ref[0]
    yc = yc_ref[0]
    zx = jnp.abs(xc - xr)
    zy = jnp.abs(yc - yr)
    zmax = jnp.maximum(zx, zy)

    inf = jnp.float32(jnp.inf)
    kk = jnp.float32(_K + 1)

    def body(_, carry):
        thresh, cum, eps = carry
        masked = jnp.where(zmax > thresh, zmax, inf)
        m = jnp.min(masked, axis=0, keepdims=True)
        mult = jnp.sum((zmax == m).astype(jnp.float32), axis=0, keepdims=True)
        take = cum < kk
        eps = jnp.where(take, m, eps)
        cum = jnp.where(take, cum + mult, cum)
        return m, cum, eps

    z0 = jnp.zeros((1, hw), jnp.float32)
    cum0 = jnp.sum((zmax == 0.0).astype(jnp.float32), axis=0, keepdims=True)
    thresh, cum, eps = jax.lax.fori_loop(0, _K - 1, body, (z0, cum0, z0))
    masked = jnp.where(zmax > thresh, zmax, inf)
    m = jnp.min(masked, axis=0, keepdims=True)
    eps = jnp.where(cum < kk, m, eps)

    cx = jnp.sum((zx < eps).astype(jnp.float32), axis=0, keepdims=True)
    cy = jnp.sum((zy < eps).astype(jnp.float32), axis=0, keepdims=True)
    t = _digamma_pos(cx) + _digamma_pos(cy)
    s = jnp.sum(t, axis=1, keepdims=True) * jnp.float32(1.0 / hw)
    out_ref[0] = s


def _tc_sums(xcol, ycol, xrow, yrow, n_rows, hw, row0):
    return pl.pallas_call(
        _mi_row_kernel,
        grid=(n_rows,),
        in_specs=[
            pl.BlockSpec((1, hw, 1), lambda i: (i + row0, 0, 0)),
            pl.BlockSpec((1, hw, 1), lambda i: (i + row0, 0, 0)),
            pl.BlockSpec((1, 1, hw), lambda i: (i + row0, 0, 0)),
            pl.BlockSpec((1, 1, hw), lambda i: (i + row0, 0, 0)),
        ],
        out_specs=pl.BlockSpec((1, 1, 1), lambda i: (i, 0, 0)),
        out_shape=jax.ShapeDtypeStruct((n_rows, 1, 1), jnp.float32),
        compiler_params=pltpu.CompilerParams(dimension_semantics=("parallel",)),
    )(xcol, ycol, xrow, yrow).reshape(n_rows)


def kernel(x, y):
    B, C, H, W = x.shape
    BC, HW = B * C, H * W
    xv = x.reshape(BC, HW)
    yv = y.reshape(BC, HW)

    parts = []
    if _SC_ROWS:
        parts.append(_sc_sums(xv, yv, HW))
    if _SC_ROWS < BC:
        xcol = xv.reshape(BC, HW, 1)
        ycol = yv.reshape(BC, HW, 1)
        xrow = xv.reshape(BC, 1, HW)
        yrow = yv.reshape(BC, 1, HW)
        parts.append(
            _tc_sums(xcol, ycol, xrow, yrow, BC - _SC_ROWS, HW, _SC_ROWS))
    sums = parts[0] if len(parts) == 1 else jnp.concatenate(parts)

    const = _jsp_digamma(jnp.float32(_K)) + _jsp_digamma(jnp.float32(HW))
    mi = const - sums.reshape(B, C)
    return jnp.maximum(mi, 0.0)


# speedup vs baseline: 24.6272x; 1.1020x over previous
import functools

import jax
import jax.numpy as jnp
from jax import lax
from jax.experimental import pallas as pl
from jax.experimental.pallas import tpu as pltpu
from jax.experimental.pallas import tpu_sc as plsc
from jax.scipy.special import digamma as _jsp_digamma

_K = 8
_L = 16
_SC_ROWS = 96


def _sc_log(w):
    bits = lax.bitcast_convert_type(w, jnp.int32)
    e = lax.shift_right_logical(bits, 23) - 127
    m = lax.bitcast_convert_type(
        jnp.bitwise_or(jnp.bitwise_and(bits, (1 << 23) - 1), 127 << 23),
        jnp.float32,
    )
    big = m > jnp.float32(1.4142135623730951)
    m = jnp.where(big, m * 0.5, m)
    e = (e + jnp.where(big, 1, 0)).astype(jnp.float32)
    s = (m - 1.0) / (m + 1.0)
    s2 = s * s
    series = 2.0 * s * (1.0 + s2 * (
        jnp.float32(1.0 / 3.0) + s2 * (
            jnp.float32(1.0 / 5.0) + s2 * jnp.float32(1.0 / 7.0))))
    return e * jnp.float32(0.6931471805599453) + series


def _sc_digamma(c):
    r = jnp.zeros_like(c)
    for i in range(8):
        r = r + 1.0 / (c + jnp.float32(i))
    w = c + jnp.float32(8.0)
    iw = 1.0 / w
    iw2 = iw * iw
    return _sc_log(w) - 0.5 * iw - iw2 * (
        jnp.float32(1.0 / 12.0)
        - iw2 * (jnp.float32(1.0 / 120.0) - iw2 * jnp.float32(1.0 / 252.0))
    ) - r


_PTS = 4


def _sc_body(nc, rpw, hw, x_hbm, y_hbm, out_hbm, xv, yv, outv):
    nv = hw // _L
    wid = lax.axis_index("s") * nc + lax.axis_index("c")
    inf16 = jnp.full((_L,), jnp.inf, jnp.float32)
    z16 = jnp.zeros((_L,), jnp.float32)
    nsub = _L // _PTS

    for r in range(rpw):
        row = wid * rpw + r
        pltpu.sync_copy(x_hbm.at[row], xv)
        pltpu.sync_copy(y_hbm.at[row], yv)

        def super_body(sb, acc):
            cxa = z16
            cya = z16
            xsb = xv[pl.ds(sb * _L, _L)]
            ysb = yv[pl.ds(sb * _L, _L)]
            for s in range(nsub):
                xs = [xsb[s * _PTS + i] for i in range(_PTS)]
                ys = [ysb[s * _PTS + i] for i in range(_PTS)]

                def dist_body(jg, tops):
                    xj = xv[pl.ds(jg * _L, _L)]
                    yj = yv[pl.ds(jg * _L, _L)]
                    new = []
                    for i in range(_PTS):
                        t = jnp.maximum(jnp.abs(xj - xs[i]),
                                        jnp.abs(yj - ys[i]))
                        lo = jnp.minimum(plsc.sort_key_val(t, t)[0], tops[i])
                        new.append(plsc.sort_key_val(lo, lo,
                                                     descending=True)[0])
                    return tuple(new)

                tops = lax.fori_loop(0, nv, dist_body, (inf16,) * _PTS)
                eps = [tops[i][_L - 1 - _K] for i in range(_PTS)]

                def cnt_body(jg, carry):
                    xj = xv[pl.ds(jg * _L, _L)]
                    yj = yv[pl.ds(jg * _L, _L)]
                    out = []
                    for i in range(_PTS):
                        ax, ay = carry[2 * i], carry[2 * i + 1]
                        out.append(ax + jnp.where(
                            jnp.abs(xj - xs[i]) < eps[i], 1.0, 0.0))
                        out.append(ay + jnp.where(
                            jnp.abs(yj - ys[i]) < eps[i], 1.0, 0.0))
                    return tuple(out)

                cnts = lax.fori_loop(0, nv, cnt_body, (z16,) * (2 * _PTS))
                for i in range(_PTS):
                    onehot = (lax.iota(jnp.int32, _L) == s * _PTS + i)
                    cxa = cxa + jnp.where(onehot, jnp.sum(cnts[2 * i]), 0.0)
                    cya = cya + jnp.where(onehot, jnp.sum(cnts[2 * i + 1]),
                                          0.0)
            return acc + _sc_digamma(cxa) + _sc_digamma(cya)

        acc = lax.fori_loop(0, nv, super_body, z16)
        outv[pl.ds(r * _L, _L)] = acc * jnp.float32(1.0 / hw)

    pltpu.sync_copy(outv, out_hbm.at[wid])


def _sc_sums(xv2, yv2, hw):
    info = plsc.get_sparse_core_info()
    nc, ns = info.num_cores, info.num_subcores
    nw = nc * ns
    rpw = _SC_ROWS // nw
    mesh = plsc.VectorSubcoreMesh(core_axis_name="c", subcore_axis_name="s")
    body = functools.partial(_sc_body, nc, rpw, hw)
    out = pl.kernel(
        body,
        out_type=jax.ShapeDtypeStruct((nw, rpw * _L), jnp.float32),
        mesh=mesh,
        scratch_types=[
            pltpu.VMEM((hw,), jnp.float32),
            pltpu.VMEM((hw,), jnp.float32),
            pltpu.VMEM((rpw * _L,), jnp.float32),
        ],
        compiler_params=pltpu.CompilerParams(needs_layout_passes=False),
    )(xv2, yv2)
    return out.reshape(_SC_ROWS, _L).sum(axis=-1)


def _digamma_pos(c):
    r = jnp.zeros_like(c)
    for i in range(8):
        r = r + 1.0 / (c + jnp.float32(i))
    w = c + jnp.float32(8.0)
    iw = 1.0 / w
    iw2 = iw * iw
    psi_w = jnp.log(w) - 0.5 * iw - iw2 * (
        jnp.float32(1.0 / 12.0)
        - iw2 * (jnp.float32(1.0 / 120.0) - iw2 * jnp.float32(1.0 / 252.0))
    )
    return psi_w - r


def _mi_row_kernel(xc_ref, yc_ref, xr_ref, yr_ref, out_ref):
    hw = xr_ref.shape[-1]
    xr = xr_ref[0]
    yr = yr_ref[0]
    xc = xc_ref[0]
    yc = yc_ref[0]
    zx = jnp.abs(xc - xr)
    zy = jnp.abs(yc - yr)
    zmax = jnp.maximum(zx, zy)

    inf = jnp.float32(jnp.inf)
    kk = jnp.float32(_K + 1)

    def body(_, carry):
        thresh, cum, eps = carry
        masked = jnp.where(zmax > thresh, zmax, inf)
        m = jnp.min(masked, axis=0, keepdims=True)
        mult = jnp.sum((zmax == m).astype(jnp.float32), axis=0, keepdims=True)
        take = cum < kk
        eps = jnp.where(take, m, eps)
        cum = jnp.where(take, cum + mult, cum)
        return m, cum, eps

    z0 = jnp.zeros((1, hw), jnp.float32)
    cum0 = jnp.sum((zmax == 0.0).astype(jnp.float32), axis=0, keepdims=True)
    thresh, cum, eps = jax.lax.fori_loop(0, _K - 1, body, (z0, cum0, z0))
    masked = jnp.where(zmax > thresh, zmax, inf)
    m = jnp.min(masked, axis=0, keepdims=True)
    eps = jnp.where(cum < kk, m, eps)

    cx = jnp.sum((zx < eps).astype(jnp.float32), axis=0, keepdims=True)
    cy = jnp.sum((zy < eps).astype(jnp.float32), axis=0, keepdims=True)
    t = _digamma_pos(cx) + _digamma_pos(cy)
    s = jnp.sum(t, axis=1, keepdims=True) * jnp.float32(1.0 / hw)
    out_ref[0] = s


def _tc_sums(xcol, ycol, xrow, yrow, n_rows, hw, row0):
    return pl.pallas_call(
        _mi_row_kernel,
        grid=(n_rows,),
        in_specs=[
            pl.BlockSpec((1, hw, 1), lambda i: (i + row0, 0, 0)),
            pl.BlockSpec((1, hw, 1), lambda i: (i + row0, 0, 0)),
            pl.BlockSpec((1, 1, hw), lambda i: (i + row0, 0, 0)),
            pl.BlockSpec((1, 1, hw), lambda i: (i + row0, 0, 0)),
        ],
        out_specs=pl.BlockSpec((1, 1, 1), lambda i: (i, 0, 0)),
        out_shape=jax.ShapeDtypeStruct((n_rows, 1, 1), jnp.float32),
        compiler_params=pltpu.CompilerParams(dimension_semantics=("parallel",)),
    )(xcol, ycol, xrow, yrow).reshape(n_rows)


def kernel(x, y):
    B, C, H, W = x.shape
    BC, HW = B * C, H * W
    xv = x.reshape(BC, HW)
    yv = y.reshape(BC, HW)

    parts = []
    if _SC_ROWS:
        parts.append(_sc_sums(xv, yv, HW))
    if _SC_ROWS < BC:
        xcol = xv.reshape(BC, HW, 1)
        ycol = yv.reshape(BC, HW, 1)
        xrow = xv.reshape(BC, 1, HW)
        yrow = yv.reshape(BC, 1, HW)
        parts.append(
            _tc_sums(xcol, ycol, xrow, yrow, BC - _SC_ROWS, HW, _SC_ROWS))
    sums = parts[0] if len(parts) == 1 else jnp.concatenate(parts)

    const = _jsp_digamma(jnp.float32(_K)) + _jsp_digamma(jnp.float32(HW))
    mi = const - sums.reshape(B, C)
    return jnp.maximum(mi, 0.0)


# speedup vs baseline: 30.2501x; 1.2283x over previous
import functools

import jax
import jax.numpy as jnp
from jax import lax
from jax.experimental import pallas as pl
from jax.experimental.pallas import tpu as pltpu
from jax.experimental.pallas import tpu_sc as plsc
from jax.scipy.special import digamma as _jsp_digamma

_K = 8
_L = 16
_SC_ROWS = 96


def _sc_log(w):
    bits = lax.bitcast_convert_type(w, jnp.int32)
    e = lax.shift_right_logical(bits, 23) - 127
    m = lax.bitcast_convert_type(
        jnp.bitwise_or(jnp.bitwise_and(bits, (1 << 23) - 1), 127 << 23),
        jnp.float32,
    )
    big = m > jnp.float32(1.4142135623730951)
    m = jnp.where(big, m * 0.5, m)
    e = (e + jnp.where(big, 1, 0)).astype(jnp.float32)
    s = (m - 1.0) / (m + 1.0)
    s2 = s * s
    series = 2.0 * s * (1.0 + s2 * (
        jnp.float32(1.0 / 3.0) + s2 * (
            jnp.float32(1.0 / 5.0) + s2 * jnp.float32(1.0 / 7.0))))
    return e * jnp.float32(0.6931471805599453) + series


def _sc_digamma(c):
    r = jnp.zeros_like(c)
    for i in range(8):
        r = r + 1.0 / (c + jnp.float32(i))
    w = c + jnp.float32(8.0)
    iw = 1.0 / w
    iw2 = iw * iw
    return _sc_log(w) - 0.5 * iw - iw2 * (
        jnp.float32(1.0 / 12.0)
        - iw2 * (jnp.float32(1.0 / 120.0) - iw2 * jnp.float32(1.0 / 252.0))
    ) - r


_PTS = 8


def _sc_body(nc, rpw, hw, x_hbm, y_hbm, out_hbm, xv, yv, outv):
    nv = hw // _L
    wid = lax.axis_index("s") * nc + lax.axis_index("c")
    inf16 = jnp.full((_L,), jnp.inf, jnp.float32)
    z16 = jnp.zeros((_L,), jnp.float32)
    nsub = _L // _PTS

    for r in range(rpw):
        row = wid * rpw + r
        pltpu.sync_copy(x_hbm.at[row], xv)
        pltpu.sync_copy(y_hbm.at[row], yv)

        def super_body(sb, acc):
            cxa = z16
            cya = z16
            xsb = xv[pl.ds(sb * _L, _L)]
            ysb = yv[pl.ds(sb * _L, _L)]
            for s in range(nsub):
                xs = [xsb[s * _PTS + i] for i in range(_PTS)]
                ys = [ysb[s * _PTS + i] for i in range(_PTS)]

                def dist_body(jg, tops):
                    xj = xv[pl.ds(jg * _L, _L)]
                    yj = yv[pl.ds(jg * _L, _L)]
                    new = []
                    for i in range(_PTS):
                        t = jnp.maximum(jnp.abs(xj - xs[i]),
                                        jnp.abs(yj - ys[i]))
                        lo = jnp.minimum(plsc.sort_key_val(t, t)[0], tops[i])
                        new.append(plsc.sort_key_val(lo, lo,
                                                     descending=True)[0])
                    return tuple(new)

                tops = lax.fori_loop(0, nv, dist_body, (inf16,) * _PTS)
                eps = [tops[i][_L - 1 - _K] for i in range(_PTS)]

                def cnt_body(jg, carry):
                    xj = xv[pl.ds(jg * _L, _L)]
                    yj = yv[pl.ds(jg * _L, _L)]
                    out = []
                    for i in range(_PTS):
                        ax, ay = carry[2 * i], carry[2 * i + 1]
                        out.append(ax + jnp.where(
                            jnp.abs(xj - xs[i]) < eps[i], 1.0, 0.0))
                        out.append(ay + jnp.where(
                            jnp.abs(yj - ys[i]) < eps[i], 1.0, 0.0))
                    return tuple(out)

                cnts = lax.fori_loop(0, nv, cnt_body, (z16,) * (2 * _PTS))
                for i in range(_PTS):
                    onehot = (lax.iota(jnp.int32, _L) == s * _PTS + i)
                    cxa = cxa + jnp.where(onehot, jnp.sum(cnts[2 * i]), 0.0)
                    cya = cya + jnp.where(onehot, jnp.sum(cnts[2 * i + 1]),
                                          0.0)
            return acc + _sc_digamma(cxa) + _sc_digamma(cya)

        acc = lax.fori_loop(0, nv, super_body, z16)
        outv[pl.ds(r * _L, _L)] = acc * jnp.float32(1.0 / hw)

    pltpu.sync_copy(outv, out_hbm.at[wid])


def _sc_sums(xv2, yv2, hw):
    info = plsc.get_sparse_core_info()
    nc, ns = info.num_cores, info.num_subcores
    nw = nc * ns
    rpw = _SC_ROWS // nw
    mesh = plsc.VectorSubcoreMesh(core_axis_name="c", subcore_axis_name="s")
    body = functools.partial(_sc_body, nc, rpw, hw)
    out = pl.kernel(
        body,
        out_type=jax.ShapeDtypeStruct((nw, rpw * _L), jnp.float32),
        mesh=mesh,
        scratch_types=[
            pltpu.VMEM((hw,), jnp.float32),
            pltpu.VMEM((hw,), jnp.float32),
            pltpu.VMEM((rpw * _L,), jnp.float32),
        ],
        compiler_params=pltpu.CompilerParams(needs_layout_passes=False),
    )(xv2, yv2)
    return out.reshape(_SC_ROWS, _L).sum(axis=-1)


def _digamma_pos(c):
    r = jnp.zeros_like(c)
    for i in range(8):
        r = r + 1.0 / (c + jnp.float32(i))
    w = c + jnp.float32(8.0)
    iw = 1.0 / w
    iw2 = iw * iw
    psi_w = jnp.log(w) - 0.5 * iw - iw2 * (
        jnp.float32(1.0 / 12.0)
        - iw2 * (jnp.float32(1.0 / 120.0) - iw2 * jnp.float32(1.0 / 252.0))
    )
    return psi_w - r


def _mi_row_kernel(xc_ref, yc_ref, xr_ref, yr_ref, out_ref):
    hw = xr_ref.shape[-1]
    xr = xr_ref[0]
    yr = yr_ref[0]
    xc = xc_ref[0]
    yc = yc_ref[0]
    zx = jnp.abs(xc - xr)
    zy = jnp.abs(yc - yr)
    zmax = jnp.maximum(zx, zy)

    inf = jnp.float32(jnp.inf)
    kk = jnp.float32(_K + 1)

    def body(_, carry):
        thresh, cum, eps = carry
        masked = jnp.where(zmax > thresh, zmax, inf)
        m = jnp.min(masked, axis=0, keepdims=True)
        mult = jnp.sum((zmax == m).astype(jnp.float32), axis=0, keepdims=True)
        take = cum < kk
        eps = jnp.where(take, m, eps)
        cum = jnp.where(take, cum + mult, cum)
        return m, cum, eps

    z0 = jnp.zeros((1, hw), jnp.float32)
    cum0 = jnp.sum((zmax == 0.0).astype(jnp.float32), axis=0, keepdims=True)
    thresh, cum, eps = jax.lax.fori_loop(0, _K - 1, body, (z0, cum0, z0))
    masked = jnp.where(zmax > thresh, zmax, inf)
    m = jnp.min(masked, axis=0, keepdims=True)
    eps = jnp.where(cum < kk, m, eps)

    cx = jnp.sum((zx < eps).astype(jnp.float32), axis=0, keepdims=True)
    cy = jnp.sum((zy < eps).astype(jnp.float32), axis=0, keepdims=True)
    t = _digamma_pos(cx) + _digamma_pos(cy)
    s = jnp.sum(t, axis=1, keepdims=True) * jnp.float32(1.0 / hw)
    out_ref[0] = s


def _tc_sums(xcol, ycol, xrow, yrow, n_rows, hw, row0):
    return pl.pallas_call(
        _mi_row_kernel,
        grid=(n_rows,),
        in_specs=[
            pl.BlockSpec((1, hw, 1), lambda i: (i + row0, 0, 0)),
            pl.BlockSpec((1, hw, 1), lambda i: (i + row0, 0, 0)),
            pl.BlockSpec((1, 1, hw), lambda i: (i + row0, 0, 0)),
            pl.BlockSpec((1, 1, hw), lambda i: (i + row0, 0, 0)),
        ],
        out_specs=pl.BlockSpec((1, 1, 1), lambda i: (i, 0, 0)),
        out_shape=jax.ShapeDtypeStruct((n_rows, 1, 1), jnp.float32),
        compiler_params=pltpu.CompilerParams(dimension_semantics=("parallel",)),
    )(xcol, ycol, xrow, yrow).reshape(n_rows)


def kernel(x, y):
    B, C, H, W = x.shape
    BC, HW = B * C, H * W
    xv = x.reshape(BC, HW)
    yv = y.reshape(BC, HW)

    parts = []
    if _SC_ROWS:
        parts.append(_sc_sums(xv, yv, HW))
    if _SC_ROWS < BC:
        xcol = xv.reshape(BC, HW, 1)
        ycol = yv.reshape(BC, HW, 1)
        xrow = xv.reshape(BC, 1, HW)
        yrow = yv.reshape(BC, 1, HW)
        parts.append(
            _tc_sums(xcol, ycol, xrow, yrow, BC - _SC_ROWS, HW, _SC_ROWS))
    sums = parts[0] if len(parts) == 1 else jnp.concatenate(parts)

    const = _jsp_digamma(jnp.float32(_K)) + _jsp_digamma(jnp.float32(HW))
    mi = const - sums.reshape(B, C)
    return jnp.maximum(mi, 0.0)
